# baseline (speedup 1.0000x reference)
import functools

import jax
import jax.numpy as jnp
from jax import lax
from jax.experimental import pallas as pl
from jax.experimental.pallas import tpu as pltpu
from jax.experimental.pallas import tpu_sc as plsc

B = 1024
L = 200
D = 128
N = B * L
NW = 32
PER_W = N // NW
CH = 128
NCH = PER_W // CH
NBUF = 5
LANES = 16


@jax.jit
def _sc_embed(ids3, table, pos):
    mesh = plsc.VectorSubcoreMesh(core_axis_name="c", subcore_axis_name="s")

    @functools.partial(
        pl.kernel,
        mesh=mesh,
        out_type=jax.ShapeDtypeStruct((N, D), jnp.float32),
        scratch_types=[
            pltpu.VMEM((NCH, CH), jnp.int32),
            pltpu.VMEM((L, D), jnp.float32),
        ]
        + [pltpu.VMEM((CH, D), jnp.float32) for _ in range(NBUF)]
        + [pltpu.SemaphoreType.DMA for _ in range(2 * NBUF)],
    )
    def k(ids_hbm, table_hbm, pos_hbm, out_hbm, idx_v, pos_v, *bufsem):
        bufs = bufsem[:NBUF]
        gsems = bufsem[NBUF:2 * NBUF]
        ssems = bufsem[2 * NBUF:]
        wid = lax.axis_index("s") * 2 + lax.axis_index("c")
        base = wid * PER_W
        pltpu.sync_copy(ids_hbm.at[wid], idx_v)
        pltpu.sync_copy(pos_hbm.at[pl.ds(0, L)], pos_v)

        def prefill(buf, c):
            p0 = lax.rem(c * CH, L)

            def row_body(r, p, buf=buf):
                for kk in range(D // LANES):
                    sl = pl.ds(kk * LANES, LANES)
                    buf[r, sl] = pos_v[p, sl]
                p = p + 1
                return jnp.where(p >= L, p - L, p)

            lax.fori_loop(0, CH, row_body, p0)

        for b in range(NBUF - 1):
            prefill(bufs[b], b)
            pltpu.async_copy(
                table_hbm.at[idx_v.at[b]], bufs[b], gsems[b], add=True
            )

        def super_body(s, _):
            c0 = s * NBUF
            for b in range(NBUF):
                c = c0 + b
                buf = bufs[b]
                pltpu.make_async_copy(
                    table_hbm.at[idx_v.at[c]], buf, gsems[b]
                ).wait()

                pltpu.async_copy(
                    buf, out_hbm.at[pl.ds(base + c * CH, CH)], ssems[b]
                )

                bn = (b + NBUF - 1) % NBUF
                cn = c + NBUF - 1

                @pl.when(c >= 1)
                def _():
                    pltpu.make_async_copy(
                        bufs[bn], out_hbm.at[pl.ds(0, CH)], ssems[bn]
                    ).wait()

                @pl.when(cn < NCH)
                def _():
                    prefill(bufs[bn], cn)
                    pltpu.async_copy(
                        table_hbm.at[idx_v.at[cn]], bufs[bn], gsems[bn],
                        add=True,
                    )

            return 0

        lax.fori_loop(0, NCH // NBUF, super_body, 0)

        bl = (NCH - 1) % NBUF
        pltpu.make_async_copy(
            bufs[bl], out_hbm.at[pl.ds(0, CH)], ssems[bl]
        ).wait()

    return k(ids3, table, pos)


def kernel(input_ids, aug_embeddings, position_embedding):
    ids3 = input_ids.astype(jnp.int32).reshape(NW, NCH, CH)
    out = _sc_embed(ids3, aug_embeddings, position_embedding)
    return out.reshape(B, L, D)


# speedup vs baseline: 2.1158x; 2.1158x over previous
import functools

import jax
import jax.numpy as jnp
from jax import lax
from jax.experimental import pallas as pl
from jax.experimental.pallas import tpu as pltpu
from jax.experimental.pallas import tpu_sc as plsc

B = 1024
L = 200
D = 128
N = B * L
NW = 32
PER_W = N // NW
CH = 80
NCH = PER_W // CH
NBUF = 5
LANES = 16


def _pos_segments(slot):
    poff = (slot * CH) % L
    segs = [(0, min(CH, L - poff), poff)]
    if L - poff < CH:
        segs.append((L - poff, CH, poff - L))
    return segs


@jax.jit
def _sc_embed(ids3, table, pos):
    mesh = plsc.VectorSubcoreMesh(core_axis_name="c", subcore_axis_name="s")

    @functools.partial(
        pl.kernel,
        mesh=mesh,
        out_type=jax.ShapeDtypeStruct((N, D), jnp.float32),
        scratch_types=[
            pltpu.VMEM((NCH, CH), jnp.int32),
            pltpu.VMEM((L, D), jnp.float32),
        ]
        + [pltpu.VMEM((CH, D), jnp.float32) for _ in range(NBUF)]
        + [pltpu.SemaphoreType.DMA for _ in range(2 * NBUF)],
    )
    def k(ids_hbm, table_hbm, pos_hbm, out_hbm, idx_v, pos_v, *bufsem):
        bufs = bufsem[:NBUF]
        gsems = bufsem[NBUF:2 * NBUF]
        ssems = bufsem[2 * NBUF:]
        wid = lax.axis_index("s") * 2 + lax.axis_index("c")
        base = wid * PER_W
        pltpu.sync_copy(ids_hbm.at[wid], idx_v)
        pltpu.sync_copy(pos_hbm.at[pl.ds(0, L)], pos_v)

        def prefill(slot):
            for lo, hi, off in _pos_segments(slot):

                def row_body(r, _, slot=slot, off=off):
                    for kk in range(D // LANES):
                        sl = pl.ds(kk * LANES, LANES)
                        bufs[slot][r, sl] = pos_v[r + off, sl]
                    return 0

                lax.fori_loop(lo, hi, row_body, 0)

        for b in range(NBUF - 1):
            prefill(b)
            pltpu.async_copy(
                table_hbm.at[idx_v.at[b]], bufs[b], gsems[b], add=True
            )

        def super_body(s, _):
            c0 = s * NBUF
            for b in range(NBUF):
                c = c0 + b
                buf = bufs[b]
                pltpu.make_async_copy(
                    table_hbm.at[idx_v.at[c]], buf, gsems[b]
                ).wait()

                pltpu.async_copy(
                    buf, out_hbm.at[pl.ds(base + c * CH, CH)], ssems[b]
                )

                bn = (b + NBUF - 1) % NBUF
                cn = c + NBUF - 1

                @pl.when(c >= 1)
                def _():
                    pltpu.make_async_copy(
                        bufs[bn], out_hbm.at[pl.ds(0, CH)], ssems[bn]
                    ).wait()

                @pl.when(cn < NCH)
                def _():
                    prefill(bn)
                    pltpu.async_copy(
                        table_hbm.at[idx_v.at[cn]], bufs[bn], gsems[bn],
                        add=True,
                    )

            return 0

        lax.fori_loop(0, NCH // NBUF, super_body, 0)

        bl = (NCH - 1) % NBUF
        pltpu.make_async_copy(
            bufs[bl], out_hbm.at[pl.ds(0, CH)], ssems[bl]
        ).wait()

    return k(ids3, table, pos)


def kernel(input_ids, aug_embeddings, position_embedding):
    ids3 = input_ids.astype(jnp.int32).reshape(NW, NCH, CH)
    out = _sc_embed(ids3, aug_embeddings, position_embedding)
    return out.reshape(B, L, D)
